# baseline (speedup 1.0000x reference)
import functools

import jax
import jax.numpy as jnp
from jax import lax
from jax.experimental import pallas as pl
from jax.experimental.pallas import tpu as pltpu
from jax.experimental.pallas import tpu_sc as plsc

_B, _S, _D = 4, 8192, 1024
_NC, _NS, _L = 2, 16, 16
_NW = _NC * _NS
_S_W = _S // _NW
_T = 32


_mesh = plsc.VectorSubcoreMesh(core_axis_name="c", subcore_axis_name="s")


@functools.partial(
    pl.kernel,
    mesh=_mesh,
    out_type=jax.ShapeDtypeStruct((_B, _S, _D), jnp.float32),
    scratch_types=[
        pltpu.VMEM((_T, _D), jnp.float32),
        pltpu.VMEM((_T, _D), jnp.float32),
    ],
)
def _sc_add(in_hbm, emb_hbm, out_hbm, emb_v, io_v):
    wid = lax.axis_index("s") * _NC + lax.axis_index("c")
    s_base = wid * _S_W

    def tile_body(t, _):
        s0 = s_base + t * _T
        pltpu.sync_copy(emb_hbm.at[pl.ds(s0, _T)], emb_v)

        def batch_body(b, _):
            pltpu.sync_copy(in_hbm.at[b, pl.ds(s0, _T)], io_v)

            def row_body(r, _):
                for c in range(_D // _L):
                    sl = pl.ds(c * _L, _L)
                    io_v[r, sl] = io_v[r, sl] + emb_v[r, sl]
                return 0

            lax.fori_loop(0, _T, row_body, 0)
            pltpu.sync_copy(io_v, out_hbm.at[b, pl.ds(s0, _T)])
            return 0

        lax.fori_loop(0, _B, batch_body, 0)
        return 0

    lax.fori_loop(0, _S_W // _T, tile_body, 0)


def kernel(inputs, embeddings):
    seq_len = inputs.shape[1]
    return _sc_add(inputs, embeddings[:seq_len])


# speedup vs baseline: 1.8207x; 1.8207x over previous
import functools

import jax
import jax.numpy as jnp
from jax import lax
from jax.experimental import pallas as pl
from jax.experimental.pallas import tpu as pltpu
from jax.experimental.pallas import tpu_sc as plsc

_B, _S, _D = 4, 8192, 1024
_NC, _NS, _L = 2, 16, 16
_NW = _NC * _NS
_S_W = _S // _NW
_TB = 8
_NT = _S_W // _TB


_mesh = plsc.VectorSubcoreMesh(core_axis_name="c", subcore_axis_name="s")


@functools.partial(
    pl.kernel,
    mesh=_mesh,
    out_type=jax.ShapeDtypeStruct((_B, _S, _D), jnp.float32),
    scratch_types=[
        pltpu.VMEM((_TB, _D), jnp.float32),
        pltpu.VMEM((_TB, _D), jnp.float32),
        pltpu.VMEM((_B, _TB, _D), jnp.float32),
        pltpu.VMEM((_B, _TB, _D), jnp.float32),
        pltpu.SemaphoreType.DMA,
        pltpu.SemaphoreType.DMA,
        pltpu.SemaphoreType.DMA,
        pltpu.SemaphoreType.DMA,
        pltpu.SemaphoreType.DMA,
        pltpu.SemaphoreType.DMA,
    ],
)
def _sc_add(in_hbm, emb_hbm, out_hbm, emb0, emb1, io0, io1,
            em_s0, em_s1, ld_s0, ld_s1, st_s0, st_s1):
    wid = lax.axis_index("s") * _NC + lax.axis_index("c")
    s_base = wid * _S_W

    embs = (emb0, emb1)
    ios = (io0, io1)
    em_sems = (em_s0, em_s1)
    ld_sems = (ld_s0, ld_s1)
    st_sems = (st_s0, st_s1)

    def emb_slice(t):
        return emb_hbm.at[pl.ds(s_base + t * _TB, _TB)]

    def in_slice(t, b):
        return in_hbm.at[b, pl.ds(s_base + t * _TB, _TB)]

    def out_slice(t, b):
        return out_hbm.at[b, pl.ds(s_base + t * _TB, _TB)]

    def start_tile_loads(t, q):
        pltpu.async_copy(emb_slice(t), embs[q], em_sems[q])
        for b in range(_B):
            pltpu.async_copy(in_slice(t, b), ios[q].at[b], ld_sems[q])

    start_tile_loads(0, 0)

    def pair_body(g, _):
        for j in (0, 1):
            t = 2 * g + j
            p, q = j, 1 - j
            emb_v, io_v = embs[p], ios[p]

            pltpu.make_async_copy(emb_slice(t), emb_v, em_sems[p]).wait()
            for b in range(_B):
                pltpu.make_async_copy(
                    in_slice(t, b), io_v.at[b], ld_sems[p]).wait()

            @pl.when(t >= 1)
            def _():
                for b in range(_B):
                    pltpu.make_async_copy(
                        ios[q].at[b], out_slice(t - 1, b), st_sems[q]).wait()

            @pl.when(t < _NT - 1)
            def _():
                start_tile_loads(t + 1, q)

            def row_body(r, _):
                for c in range(_D // _L):
                    sl = pl.ds(c * _L, _L)
                    e = emb_v[r, sl]
                    for b in range(_B):
                        io_v[b, r, sl] = io_v[b, r, sl] + e
                return 0

            lax.fori_loop(0, _TB, row_body, 0)

            for b in range(_B):
                pltpu.async_copy(io_v.at[b], out_slice(t, b), st_sems[p])
        return 0

    lax.fori_loop(0, _NT // 2, pair_body, 0)

    for b in range(_B):
        pltpu.make_async_copy(
            ios[1].at[b], out_slice(_NT - 1, b), st_sems[1]).wait()


def kernel(inputs, embeddings):
    seq_len = inputs.shape[1]
    return _sc_add(inputs, embeddings[:seq_len])


# speedup vs baseline: 1.9516x; 1.0719x over previous
import functools

import jax
import jax.numpy as jnp
from jax import lax
from jax.experimental import pallas as pl
from jax.experimental.pallas import tpu as pltpu
from jax.experimental.pallas import tpu_sc as plsc

_B, _S, _D = 4, 8192, 1024
_NC, _NS, _L = 2, 16, 16
_NW = _NC * _NS
_S_W = _S // _NW
_TB = 8
_NT = _S_W // _TB


_mesh = plsc.VectorSubcoreMesh(core_axis_name="c", subcore_axis_name="s")


@functools.partial(
    pl.kernel,
    mesh=_mesh,
    out_type=jax.ShapeDtypeStruct((_B, _S, _D), jnp.float32),
    scratch_types=[
        pltpu.VMEM((_TB, _D), jnp.float32),
        pltpu.VMEM((_TB, _D), jnp.float32),
        pltpu.VMEM((_B, _TB, _D), jnp.float32),
        pltpu.VMEM((_B, _TB, _D), jnp.float32),
        pltpu.SemaphoreType.DMA,
        pltpu.SemaphoreType.DMA,
        pltpu.SemaphoreType.DMA,
        pltpu.SemaphoreType.DMA,
        pltpu.SemaphoreType.DMA,
        pltpu.SemaphoreType.DMA,
    ],
)
def _sc_add(in_hbm, emb_hbm, out_hbm, emb0, emb1, io0, io1,
            em_s0, em_s1, ld_s0, ld_s1, st_s0, st_s1):
    wid = lax.axis_index("s") * _NC + lax.axis_index("c")
    s_base = wid * _S_W

    embs = (emb0, emb1)
    ios = (io0, io1)
    em_sems = (em_s0, em_s1)
    ld_sems = (ld_s0, ld_s1)
    st_sems = (st_s0, st_s1)

    def emb_slice(t):
        return emb_hbm.at[pl.ds(s_base + t * _TB, _TB)]

    def in_slice(t, b):
        return in_hbm.at[b, pl.ds(s_base + t * _TB, _TB)]

    def out_slice(t, b):
        return out_hbm.at[b, pl.ds(s_base + t * _TB, _TB)]

    def start_tile_loads(t, q):
        pltpu.async_copy(emb_slice(t), embs[q], em_sems[q])
        for b in range(_B):
            pltpu.async_copy(in_slice(t, b), ios[q].at[b], ld_sems[q])

    start_tile_loads(0, 0)

    def pair_body(g, _):
        for j in (0, 1):
            t = 2 * g + j
            p, q = j, 1 - j
            emb_v, io_v = embs[p], ios[p]

            pltpu.make_async_copy(emb_slice(t), emb_v, em_sems[p]).wait()
            for b in range(_B):
                pltpu.make_async_copy(
                    in_slice(t, b), io_v.at[b], ld_sems[p]).wait()

            @pl.when(t >= 1)
            def _():
                for b in range(_B):
                    pltpu.make_async_copy(
                        ios[q].at[b], out_slice(t - 1, b), st_sems[q]).wait()

            @pl.when(t < _NT - 1)
            def _():
                start_tile_loads(t + 1, q)

            def row_body(r, _):
                for c in range(_D // _L):
                    sl = pl.ds(c * _L, _L)
                    e = emb_v[r, sl]
                    for b in range(_B):
                        plsc.addupdate(io_v.at[b, r, sl], e)
                return 0

            lax.fori_loop(0, _TB, row_body, 0)

            for b in range(_B):
                pltpu.async_copy(io_v.at[b], out_slice(t, b), st_sems[p])
        return 0

    lax.fori_loop(0, _NT // 2, pair_body, 0)

    for b in range(_B):
        pltpu.make_async_copy(
            ios[1].at[b], out_slice(_NT - 1, b), st_sems[1]).wait()


def kernel(inputs, embeddings):
    seq_len = inputs.shape[1]
    return _sc_add(inputs, embeddings[:seq_len])
